# baseline (speedup 1.0000x reference)
import functools

import jax
import jax.numpy as jnp
from jax import lax
from jax.experimental import pallas as pl
from jax.experimental.pallas import tpu as pltpu
from jax.experimental.pallas import tpu_sc as plsc

_NC = 2
_NS = 16
_NW = _NC * _NS
_CH = 128
_NB = 2
_RB = 128
_NST = 4


_DN_NT = (((1,), (1,)), ((), ()))


def _mm_body(x_ref, w1_ref, wg_ref, out_ref):
    h = jnp.maximum(
        lax.dot_general(x_ref[...], w1_ref[...], _DN_NT,
                        preferred_element_type=jnp.float32), 0.0)
    hw = lax.dot_general(h, wg_ref[...], _DN_NT,
                         preferred_element_type=jnp.float32)
    out_ref[...] = jnp.broadcast_to(hw[None], out_ref.shape)


def _make_pad_body(ept, ept_pad, n, n_pad):
    def _pad_body(conn_ref, out_ref):
        out_ref[:, 0, 0, :ept] = conn_ref[:, 0, 0, :]
        iota = lax.broadcasted_iota(jnp.int32, (1, ept_pad - ept), 1)
        out_ref[0:1, 0, 0, ept:] = iota % n
        out_ref[1:2, 0, 0, ept:] = n + iota % (n_pad - n)
    return _pad_body


def _norm_body(acc_ref, out_ref):
    s = acc_ref[0] + acc_ref[1]
    nrm = jnp.sqrt(jnp.sum(s * s, axis=1, keepdims=True))
    out_ref[...] = s / nrm


def _make_sc_scatter(n_rows, n_pad, d, n_chunks):
    mesh = plsc.VectorSubcoreMesh(core_axis_name="c", subcore_axis_name="s")
    rows_per_tile = n_pad // _NS
    n_copies = rows_per_tile // _RB

    @functools.partial(
        pl.kernel,
        mesh=mesh,
        out_type=jax.ShapeDtypeStruct((_NC, n_pad, d), jnp.float32),
        scratch_types=[
            pltpu.VMEM((n_chunks // _NST, _CH), jnp.int32),
            pltpu.VMEM((n_chunks // _NST, _CH), jnp.int32),
        ]
        + [pltpu.VMEM((_CH, d), jnp.float32) for _ in range(_NB)]
        + [pltpu.VMEM_SHARED((n_pad, d), jnp.float32)]
        + [pltpu.SemaphoreType.DMA for _ in range(2 * _NB)],
    )
    def sc_kernel(hw_hbm, src_hbm, dst_hbm, out_hbm,
                  src_v, dst_v, *rest):
        rows = rest[:_NB]
        acc = rest[_NB]
        gsem = rest[_NB + 1: 2 * _NB + 1]
        ssem = rest[2 * _NB + 1:]
        cid = lax.axis_index("c")
        sid = lax.axis_index("s")
        wid = cid * _NS + sid

        zvec = jnp.zeros((16,), jnp.float32)

        def zstore(i, _):
            r = i // (d // 16)
            l = (i % (d // 16)) * 16
            rows[0][r, pl.ds(l, 16)] = zvec
            return 0

        lax.fori_loop(0, _RB * (d // 16), zstore, 0)

        def zcopy(k, _):
            pltpu.sync_copy(rows[0].at[pl.ds(0, _RB)],
                            acc.at[pl.ds(sid * rows_per_tile + k * _RB, _RB)])
            return 0

        lax.fori_loop(0, n_copies, zcopy, 0)
        plsc.subcore_barrier()

        n_half = n_chunks // _NST

        hw_c = hw_hbm.at[cid]

        n_tri = n_half // _NB

        def half(h, _):
            pltpu.sync_copy(src_hbm.at[wid].at[h], src_v)
            pltpu.sync_copy(dst_hbm.at[wid].at[h], dst_v)
            for k in range(_NB):
                pltpu.async_copy(hw_c.at[src_v.at[k]], rows[k], gsem[k])

            def tri(tt, _):
                t = _NB * tt
                for k in range(_NB):
                    pltpu.make_async_copy(
                        hw_c.at[src_v.at[t + k]], rows[k], gsem[k]).wait()
                    pltpu.async_copy(
                        rows[k], acc.at[dst_v.at[t + k]], ssem[k], add=True)
                for k in range(_NB):
                    pltpu.make_async_copy(
                        rows[k], acc.at[dst_v.at[t + k]], ssem[k]).wait()
                    pltpu.async_copy(
                        hw_c.at[src_v.at[t + _NB + k]], rows[k], gsem[k])
                return 0

            lax.fori_loop(0, n_tri - 1, tri, 0)
            t_last = (n_tri - 1) * _NB
            for k in range(_NB):
                pltpu.make_async_copy(
                    hw_c.at[src_v.at[t_last + k]], rows[k], gsem[k]).wait()
                pltpu.sync_copy(rows[k], acc.at[dst_v.at[t_last + k]], add=True)
            return 0

        lax.fori_loop(0, _NST, half, 0)
        plsc.subcore_barrier()

        def wb(k, _):
            off = sid * rows_per_tile + k * _RB
            pltpu.sync_copy(acc.at[pl.ds(off, _RB)], rows[0].at[pl.ds(0, _RB)])
            pltpu.sync_copy(rows[0].at[pl.ds(0, _RB)],
                            out_hbm.at[cid].at[pl.ds(off, _RB)])
            return 0

        lax.fori_loop(0, n_copies, wb, 0)

    return sc_kernel


def kernel(x, conn, W1, Wg):
    n, d = x.shape
    e = conn.shape[1]

    blk = 1000
    n_blk = n // blk
    hw = pl.pallas_call(
        _mm_body,
        grid=(n_blk,),
        in_specs=[
            pl.BlockSpec((blk, d), lambda i: (i, 0)),
            pl.BlockSpec((d, d), lambda i: (0, 0)),
            pl.BlockSpec((d, d), lambda i: (0, 0)),
        ],
        out_specs=pl.BlockSpec((_NC, blk, d), lambda i: (0, i, 0)),
        out_shape=jax.ShapeDtypeStruct((_NC, n, d), jnp.float32),
    )(x, W1, Wg)

    ept = e // _NW
    n_chunks = -(-ept // _CH)
    n_chunks = -(-n_chunks // (_NST * _NB)) * (_NST * _NB)
    ept_pad = n_chunks * _CH
    n_pad = -(-(n + 1) // (_NS * _RB)) * (_NS * _RB)
    conn_pad = pl.pallas_call(
        _make_pad_body(ept, ept_pad, n, n_pad),
        grid=(_NW,),
        in_specs=[pl.BlockSpec((2, 1, 1, ept), lambda t: (0, t, 0, 0))],
        out_specs=pl.BlockSpec((2, 1, 1, ept_pad), lambda t: (0, t, 0, 0)),
        out_shape=jax.ShapeDtypeStruct((2, _NW, 1, ept_pad), jnp.int32),
    )(conn.reshape(2, _NW, 1, ept))
    srcb = conn_pad[0].reshape(_NW, _NST, n_chunks // _NST, _CH)
    dstb = conn_pad[1].reshape(_NW, _NST, n_chunks // _NST, _CH)
    partials = _make_sc_scatter(n, n_pad, d, n_chunks)(hw, srcb, dstb)

    out = pl.pallas_call(
        _norm_body,
        grid=(n_blk,),
        in_specs=[pl.BlockSpec((_NC, blk, d), lambda i: (0, i, 0))],
        out_specs=pl.BlockSpec((blk, d), lambda i: (i, 0)),
        out_shape=jax.ShapeDtypeStruct((n, d), jnp.float32),
    )(partials)
    return out


# speedup vs baseline: 1.1261x; 1.1261x over previous
import functools

import jax
import jax.numpy as jnp
from jax import lax
from jax.experimental import pallas as pl
from jax.experimental.pallas import tpu as pltpu
from jax.experimental.pallas import tpu_sc as plsc

_NC = 2
_NS = 16
_NW = _NC * _NS
_CH = 128
_NB = 2
_RB = 128
_NST = 2


_DN_NT = (((1,), (1,)), ((), ()))


def _mm_body(x_ref, w1_ref, wg_ref, out_ref):
    h = jnp.maximum(
        lax.dot_general(x_ref[...], w1_ref[...], _DN_NT,
                        preferred_element_type=jnp.float32), 0.0)
    hw = lax.dot_general(h, wg_ref[...], _DN_NT,
                         preferred_element_type=jnp.float32)
    out_ref[...] = jnp.broadcast_to(hw[None], out_ref.shape)


def _make_pad_body(ept, ept_pad, n, n_pad):
    def _pad_body(conn_ref, out_ref):
        out_ref[:, 0, 0, :ept] = conn_ref[:, 0, 0, :]
        iota = lax.broadcasted_iota(jnp.int32, (1, ept_pad - ept), 1)
        out_ref[0:1, 0, 0, ept:] = iota % n
        out_ref[1:2, 0, 0, ept:] = n + iota % (n_pad - n)
    return _pad_body


def _norm_body(acc_ref, out_ref):
    s = acc_ref[0] + acc_ref[1]
    nrm = jnp.sqrt(jnp.sum(s * s, axis=1, keepdims=True))
    out_ref[...] = s / nrm


def _make_sc_scatter(n_rows, n_pad, d, n_chunks):
    mesh = plsc.VectorSubcoreMesh(core_axis_name="c", subcore_axis_name="s")
    rows_per_tile = n_pad // _NS
    n_copies = rows_per_tile // _RB

    @functools.partial(
        pl.kernel,
        mesh=mesh,
        out_type=jax.ShapeDtypeStruct((_NC, n_pad, d), jnp.float32),
        scratch_types=[
            pltpu.VMEM((n_chunks // _NST, _CH), jnp.int32),
            pltpu.VMEM((n_chunks // _NST, _CH), jnp.int32),
        ]
        + [pltpu.VMEM((_CH, d), jnp.float32) for _ in range(_NB)]
        + [pltpu.VMEM_SHARED((n_pad, d), jnp.float32)]
        + [pltpu.SemaphoreType.DMA for _ in range(2 * _NB)],
    )
    def sc_kernel(hw_hbm, src_hbm, dst_hbm, out_hbm,
                  src_v, dst_v, *rest):
        rows = rest[:_NB]
        acc = rest[_NB]
        gsem = rest[_NB + 1: 2 * _NB + 1]
        ssem = rest[2 * _NB + 1:]
        cid = lax.axis_index("c")
        sid = lax.axis_index("s")
        wid = cid * _NS + sid

        zvec = jnp.zeros((16,), jnp.float32)

        def zstore(i, _):
            r = i // (d // 16)
            l = (i % (d // 16)) * 16
            rows[0][r, pl.ds(l, 16)] = zvec
            return 0

        lax.fori_loop(0, _RB * (d // 16), zstore, 0)

        def zcopy(k, _):
            pltpu.sync_copy(rows[0].at[pl.ds(0, _RB)],
                            acc.at[pl.ds(sid * rows_per_tile + k * _RB, _RB)])
            return 0

        lax.fori_loop(0, n_copies, zcopy, 0)
        plsc.subcore_barrier()

        n_half = n_chunks // _NST

        hw_c = hw_hbm.at[cid]

        n_pairs = n_half // 2

        def half(h, _):
            pltpu.sync_copy(src_hbm.at[wid].at[h], src_v)
            pltpu.sync_copy(dst_hbm.at[wid].at[h], dst_v)
            pltpu.async_copy(hw_c.at[src_v.at[0]], rows[0], gsem[0])

            def body(jj, _):
                j = 2 * jj
                pltpu.make_async_copy(hw_c.at[src_v.at[j]], rows[0], gsem[0]).wait()
                pltpu.async_copy(hw_c.at[src_v.at[j + 1]], rows[1], gsem[1])
                pltpu.sync_copy(rows[0], acc.at[dst_v.at[j]], add=True)
                pltpu.make_async_copy(
                    hw_c.at[src_v.at[j + 1]], rows[1], gsem[1]).wait()

                @pl.when(jj + 1 < n_pairs)
                def _():
                    pltpu.async_copy(hw_c.at[src_v.at[j + 2]], rows[0], gsem[0])

                pltpu.sync_copy(rows[1], acc.at[dst_v.at[j + 1]], add=True)
                return 0

            lax.fori_loop(0, n_pairs, body, 0)
            return 0

        lax.fori_loop(0, _NST, half, 0)
        plsc.subcore_barrier()

        def wb(k, _):
            off = sid * rows_per_tile + k * _RB
            pltpu.sync_copy(acc.at[pl.ds(off, _RB)], rows[0].at[pl.ds(0, _RB)])
            pltpu.sync_copy(rows[0].at[pl.ds(0, _RB)],
                            out_hbm.at[cid].at[pl.ds(off, _RB)])
            return 0

        lax.fori_loop(0, n_copies, wb, 0)

    return sc_kernel


def kernel(x, conn, W1, Wg):
    n, d = x.shape
    e = conn.shape[1]

    blk = 1000
    n_blk = n // blk
    hw = pl.pallas_call(
        _mm_body,
        grid=(n_blk,),
        in_specs=[
            pl.BlockSpec((blk, d), lambda i: (i, 0)),
            pl.BlockSpec((d, d), lambda i: (0, 0)),
            pl.BlockSpec((d, d), lambda i: (0, 0)),
        ],
        out_specs=pl.BlockSpec((_NC, blk, d), lambda i: (0, i, 0)),
        out_shape=jax.ShapeDtypeStruct((_NC, n, d), jnp.float32),
    )(x, W1, Wg)

    ept = e // _NW
    n_chunks = -(-ept // _CH)
    n_chunks = -(-n_chunks // (_NST * _NB)) * (_NST * _NB)
    ept_pad = n_chunks * _CH
    n_pad = -(-(n + 1) // (_NS * _RB)) * (_NS * _RB)
    conn_pad = pl.pallas_call(
        _make_pad_body(ept, ept_pad, n, n_pad),
        grid=(_NW,),
        in_specs=[pl.BlockSpec((2, 1, 1, ept), lambda t: (0, t, 0, 0))],
        out_specs=pl.BlockSpec((2, 1, 1, ept_pad), lambda t: (0, t, 0, 0)),
        out_shape=jax.ShapeDtypeStruct((2, _NW, 1, ept_pad), jnp.int32),
    )(conn.reshape(2, _NW, 1, ept))
    srcb = conn_pad[0].reshape(_NW, _NST, n_chunks // _NST, _CH)
    dstb = conn_pad[1].reshape(_NW, _NST, n_chunks // _NST, _CH)
    partials = _make_sc_scatter(n, n_pad, d, n_chunks)(hw, srcb, dstb)

    out = pl.pallas_call(
        _norm_body,
        grid=(n_blk,),
        in_specs=[pl.BlockSpec((_NC, blk, d), lambda i: (0, i, 0))],
        out_specs=pl.BlockSpec((blk, d), lambda i: (i, 0)),
        out_shape=jax.ShapeDtypeStruct((n, d), jnp.float32),
    )(partials)
    return out


# speedup vs baseline: 1.1336x; 1.0066x over previous
import functools

import jax
import jax.numpy as jnp
from jax import lax
from jax.experimental import pallas as pl
from jax.experimental.pallas import tpu as pltpu
from jax.experimental.pallas import tpu_sc as plsc

_NC = 2
_NS = 16
_NW = _NC * _NS
_CH = 128
_NB = 2
_RB = 128
_NST = 2


_DN_NT = (((1,), (1,)), ((), ()))


def _mm_body(x_ref, w1_ref, wg_ref, out_ref):
    h = jnp.maximum(
        lax.dot_general(x_ref[...], w1_ref[...], _DN_NT,
                        preferred_element_type=jnp.float32), 0.0)
    out_ref[...] = lax.dot_general(h, wg_ref[...], _DN_NT,
                                   preferred_element_type=jnp.float32)


def _make_pad_body(ept, ept_pad, n, n_pad):
    def _pad_body(conn_ref, out_ref):
        out_ref[:, 0, 0, :ept] = conn_ref[:, 0, 0, :]
        iota = lax.broadcasted_iota(jnp.int32, (1, ept_pad - ept), 1)
        out_ref[0:1, 0, 0, ept:] = iota % n
        out_ref[1:2, 0, 0, ept:] = n + iota % (n_pad - n)
    return _pad_body


def _norm_body(acc_ref, out_ref):
    s = acc_ref[0] + acc_ref[1]
    nrm = jnp.sqrt(jnp.sum(s * s, axis=1, keepdims=True))
    out_ref[...] = s / nrm


def _make_sc_scatter(n_rows, n_pad, d, n_chunks):
    mesh = plsc.VectorSubcoreMesh(core_axis_name="c", subcore_axis_name="s")
    rows_per_tile = n_pad // _NS
    n_copies = rows_per_tile // _RB

    @functools.partial(
        pl.kernel,
        mesh=mesh,
        out_type=jax.ShapeDtypeStruct((_NC, n_pad, d), jnp.float32),
        scratch_types=[
            pltpu.VMEM((n_chunks // _NST, _CH), jnp.int32),
            pltpu.VMEM((n_chunks // _NST, _CH), jnp.int32),
        ]
        + [pltpu.VMEM((_CH, d), jnp.float32) for _ in range(_NB)]
        + [pltpu.VMEM_SHARED((n_pad, d), jnp.float32)]
        + [pltpu.SemaphoreType.DMA for _ in range(2 * _NB)],
    )
    def sc_kernel(hw_hbm, src_hbm, dst_hbm, out_hbm,
                  src_v, dst_v, *rest):
        rows = rest[:_NB]
        acc = rest[_NB]
        gsem = rest[_NB + 1: 2 * _NB + 1]
        ssem = rest[2 * _NB + 1:]
        cid = lax.axis_index("c")
        sid = lax.axis_index("s")
        wid = cid * _NS + sid

        zvec = jnp.zeros((16,), jnp.float32)

        def zstore(i, _):
            r = i // (d // 16)
            l = (i % (d // 16)) * 16
            rows[0][r, pl.ds(l, 16)] = zvec
            return 0

        lax.fori_loop(0, _RB * (d // 16), zstore, 0)

        def zcopy(k, _):
            pltpu.sync_copy(rows[0].at[pl.ds(0, _RB)],
                            acc.at[pl.ds(sid * rows_per_tile + k * _RB, _RB)])
            return 0

        lax.fori_loop(0, n_copies, zcopy, 0)
        plsc.subcore_barrier()

        n_half = n_chunks // _NST

        hw_c = hw_hbm

        n_pairs = n_half // 2

        def half(h, _):
            pltpu.sync_copy(src_hbm.at[wid].at[h], src_v)
            pltpu.sync_copy(dst_hbm.at[wid].at[h], dst_v)
            pltpu.async_copy(hw_c.at[src_v.at[0]], rows[0], gsem[0])

            def body(jj, _):
                j = 2 * jj
                pltpu.make_async_copy(hw_c.at[src_v.at[j]], rows[0], gsem[0]).wait()
                pltpu.async_copy(hw_c.at[src_v.at[j + 1]], rows[1], gsem[1])
                pltpu.sync_copy(rows[0], acc.at[dst_v.at[j]], add=True)
                pltpu.make_async_copy(
                    hw_c.at[src_v.at[j + 1]], rows[1], gsem[1]).wait()

                @pl.when(jj + 1 < n_pairs)
                def _():
                    pltpu.async_copy(hw_c.at[src_v.at[j + 2]], rows[0], gsem[0])

                pltpu.sync_copy(rows[1], acc.at[dst_v.at[j + 1]], add=True)
                return 0

            lax.fori_loop(0, n_pairs, body, 0)
            return 0

        lax.fori_loop(0, _NST, half, 0)
        plsc.subcore_barrier()

        def wb(k, _):
            off = sid * rows_per_tile + k * _RB
            pltpu.sync_copy(acc.at[pl.ds(off, _RB)], rows[0].at[pl.ds(0, _RB)])
            pltpu.sync_copy(rows[0].at[pl.ds(0, _RB)],
                            out_hbm.at[cid].at[pl.ds(off, _RB)])
            return 0

        lax.fori_loop(0, n_copies, wb, 0)

    return sc_kernel


def kernel(x, conn, W1, Wg):
    n, d = x.shape
    e = conn.shape[1]

    blk = 1000
    n_blk = n // blk
    hw = pl.pallas_call(
        _mm_body,
        grid=(n_blk,),
        in_specs=[
            pl.BlockSpec((blk, d), lambda i: (i, 0)),
            pl.BlockSpec((d, d), lambda i: (0, 0)),
            pl.BlockSpec((d, d), lambda i: (0, 0)),
        ],
        out_specs=pl.BlockSpec((blk, d), lambda i: (i, 0)),
        out_shape=jax.ShapeDtypeStruct((n, d), jnp.float32),
    )(x, W1, Wg)

    ept = e // _NW
    n_chunks = -(-ept // _CH)
    n_chunks = -(-n_chunks // (_NST * _NB)) * (_NST * _NB)
    ept_pad = n_chunks * _CH
    n_pad = -(-(n + 1) // (_NS * _RB)) * (_NS * _RB)
    conn_pad = pl.pallas_call(
        _make_pad_body(ept, ept_pad, n, n_pad),
        grid=(_NW,),
        in_specs=[pl.BlockSpec((2, 1, 1, ept), lambda t: (0, t, 0, 0))],
        out_specs=pl.BlockSpec((2, 1, 1, ept_pad), lambda t: (0, t, 0, 0)),
        out_shape=jax.ShapeDtypeStruct((2, _NW, 1, ept_pad), jnp.int32),
    )(conn.reshape(2, _NW, 1, ept))
    srcb = conn_pad[0].reshape(_NW, _NST, n_chunks // _NST, _CH)
    dstb = conn_pad[1].reshape(_NW, _NST, n_chunks // _NST, _CH)
    partials = _make_sc_scatter(n, n_pad, d, n_chunks)(hw, srcb, dstb)

    out = pl.pallas_call(
        _norm_body,
        grid=(n_blk,),
        in_specs=[pl.BlockSpec((_NC, blk, d), lambda i: (0, i, 0))],
        out_specs=pl.BlockSpec((blk, d), lambda i: (i, 0)),
        out_shape=jax.ShapeDtypeStruct((n, d), jnp.float32),
    )(partials)
    return out
